# baseline (speedup 1.0000x reference)
import functools

import jax
import jax.numpy as jnp
from jax import lax
from jax.experimental import pallas as pl
from jax.experimental.pallas import tpu as pltpu
from jax.experimental.pallas import tpu_sc as plsc

_REPS = 8


def kernel(x, pos_emb):
    batch = x.shape[0]
    seq, emb = pos_emb.shape
    flat = seq * emb
    table = pos_emb.reshape(flat)

    info = plsc.get_sparse_core_info()
    num_workers = info.num_cores * info.num_subcores
    rows_per_w = batch // num_workers
    reps = _REPS
    while rows_per_w % reps:
        reps //= 2
    mesh = plsc.VectorSubcoreMesh(core_axis_name="c", subcore_axis_name="s")

    @functools.partial(
        pl.kernel,
        mesh=mesh,
        out_type=jax.ShapeDtypeStruct((batch, flat), pos_emb.dtype),
        scratch_types=[pltpu.VMEM((reps, flat), pos_emb.dtype)],
    )
    def sc_broadcast(table_hbm, out_hbm, buf):
        wid = lax.axis_index("s") * info.num_cores + lax.axis_index("c")
        base = wid * rows_per_w
        for r in range(reps):
            pltpu.sync_copy(table_hbm, buf.at[r])

        def body(j, carry):
            pltpu.sync_copy(buf, out_hbm.at[pl.ds(base + j * reps, reps)])
            return carry

        lax.fori_loop(0, rows_per_w // reps, body, 0)

    out = sc_broadcast(table)
    return out.reshape(batch, seq, emb)


# speedup vs baseline: 1.0014x; 1.0014x over previous
import functools

import jax
import jax.numpy as jnp
from jax import lax
from jax.experimental import pallas as pl
from jax.experimental.pallas import tpu as pltpu
from jax.experimental.pallas import tpu_sc as plsc

_REPS = 8


def kernel(x, pos_emb):
    batch = x.shape[0]
    seq, emb = pos_emb.shape
    flat = seq * emb
    table = pos_emb.reshape(flat)

    info = plsc.get_sparse_core_info()
    num_workers = info.num_cores * info.num_subcores
    rows_per_w = batch // num_workers
    reps = _REPS
    while rows_per_w % reps:
        reps //= 2
    mesh = plsc.VectorSubcoreMesh(core_axis_name="c", subcore_axis_name="s")

    n_chunks = rows_per_w // reps
    depth = min(4, n_chunks)

    @functools.partial(
        pl.kernel,
        mesh=mesh,
        out_type=jax.ShapeDtypeStruct((batch, flat), pos_emb.dtype),
        scratch_types=[
            pltpu.VMEM((reps, flat), pos_emb.dtype),
            pltpu.SemaphoreType.DMA,
        ],
    )
    def sc_broadcast(table_hbm, out_hbm, buf, sem):
        wid = lax.axis_index("s") * info.num_cores + lax.axis_index("c")
        base = wid * rows_per_w
        for r in range(reps):
            pltpu.sync_copy(table_hbm, buf.at[r])

        def chunk_copy(j):
            return pltpu.make_async_copy(
                buf, out_hbm.at[pl.ds(base + j * reps, reps)], sem
            )

        def body(j, carry):
            chunk_copy(j).start()

            @pl.when(j >= depth)
            def _():
                chunk_copy(0).wait()

            return carry

        lax.fori_loop(0, n_chunks, body, 0)
        for _ in range(depth):
            chunk_copy(0).wait()

    out = sc_broadcast(table)
    return out.reshape(batch, seq, emb)
